# baseline (speedup 1.0000x reference)
import functools

import jax
import jax.numpy as jnp
from jax import lax
from jax.experimental import pallas as pl
from jax.experimental.pallas import tpu as pltpu
from jax.experimental.pallas import tpu_sc as plsc

_B, _S, _H = 4, 8192, 1024
_N = _B * _S
_NC, _NS = 2, 16
_NW = _NC * _NS
_TOKW = _N // _NW
_CHUNK = 32
_NCH = _TOKW // _CHUNK
_LANES = 16

_mesh = plsc.VectorSubcoreMesh(core_axis_name="c", subcore_axis_name="s")


@functools.partial(
    pl.kernel,
    out_type=jax.ShapeDtypeStruct((_N, _H), jnp.float32),
    mesh=_mesh,
    scratch_types=[
        pltpu.VMEM((_NCH, _CHUNK), jnp.int32),
        pltpu.VMEM((_NCH, _CHUNK), jnp.int32),
        pltpu.VMEM((_CHUNK, _H), jnp.float32),
        pltpu.VMEM((_CHUNK, _H), jnp.float32),
        pltpu.VMEM((_CHUNK, _H), jnp.float32),
        pltpu.SemaphoreType.DMA,
        pltpu.SemaphoreType.DMA,
        pltpu.SemaphoreType.DMA,
        pltpu.SemaphoreType.DMA,
        pltpu.SemaphoreType.DMA,
    ],
)
def _embed(ids_hbm, pos_hbm, wt_hbm, pt_hbm, out_hbm,
           widx, pidx, bufw0, bufw1, bufp,
           semgw0, semgw1, semgp, semst0, semst1):
    wid = lax.axis_index("s") * _NC + lax.axis_index("c")
    pltpu.sync_copy(ids_hbm.at[wid], widx)
    pltpu.sync_copy(pos_hbm.at[wid], pidx)

    wslots = ((bufw0, semgw0, semst0), (bufw1, semgw1, semst1))
    half = _CHUNK // 2

    def segment(c, k, mode):
        bufw, semgw, semst = wslots[k]
        o_bufw, o_semgw, o_semst = wslots[1 - k]

        pltpu.make_async_copy(wt_hbm.at[pl.ds(0, _CHUNK)], bufw, semgw).wait()
        pltpu.make_async_copy(wt_hbm.at[pl.ds(0, _CHUNK)], bufp, semgp).wait()

        if mode != "first":
            pltpu.make_async_copy(
                o_bufw, out_hbm.at[pl.ds(0, _CHUNK)], o_semst).wait()

        def fire_words():
            pltpu.async_copy(wt_hbm.at[widx.at[c + 1]], o_bufw, o_semgw)

        def fire_pos():
            pltpu.async_copy(pt_hbm.at[pidx.at[c + 1]], bufp, semgp)

        if mode == "loop":
            pl.when(c + 1 < _NCH)(fire_words)
        elif c + 1 < _NCH:
            fire_words()

        row0 = wid * _TOKW + c * _CHUNK

        @pl.loop(0, half)
        def _rows_lo(r):
            for j in range(_H // _LANES):
                sl = pl.ds(j * _LANES, _LANES)
                bufw[r, sl] += bufp[r, sl]

        pltpu.async_copy(bufw.at[pl.ds(0, half)],
                         out_hbm.at[pl.ds(row0, half)], semst)

        @pl.loop(half, _CHUNK)
        def _rows_hi(r):
            for j in range(_H // _LANES):
                sl = pl.ds(j * _LANES, _LANES)
                bufw[r, sl] += bufp[r, sl]

        if mode == "loop":
            pl.when(c + 1 < _NCH)(fire_pos)
        elif c + 1 < _NCH:
            fire_pos()

        pltpu.async_copy(bufw.at[pl.ds(half, half)],
                         out_hbm.at[pl.ds(row0 + half, half)], semst)

    pltpu.async_copy(wt_hbm.at[widx.at[0]], bufw0, semgw0)
    pltpu.async_copy(pt_hbm.at[pidx.at[0]], bufp, semgp)

    segment(0, 0, mode="first")
    segment(1, 1, mode="second")

    @pl.loop(2, _NCH, step=2)
    def _ring(c0):
        for k in range(2):
            segment(c0 + k, k, mode="loop")

    pltpu.make_async_copy(bufw1, out_hbm.at[pl.ds(0, _CHUNK)], semst1).wait()


@jax.jit
def kernel(input_ids, position_ids, word_table, pos_table):
    ids = input_ids.astype(jnp.int32).reshape(_NW, _NCH, _CHUNK)
    pos = position_ids.astype(jnp.int32).reshape(_NW, _NCH, _CHUNK)
    out = _embed(ids, pos, word_table, pos_table)
    return out.reshape(_B, _S, _H)


# speedup vs baseline: 1.2899x; 1.2899x over previous
import functools

import jax
import jax.numpy as jnp
from jax import lax
from jax.experimental import pallas as pl
from jax.experimental.pallas import tpu as pltpu
from jax.experimental.pallas import tpu_sc as plsc

_B, _S, _H = 4, 8192, 1024
_N = _B * _S
_NC, _NS = 2, 16
_NW = _NC * _NS
_TOKW = _N // _NW
_CHUNK = 32
_HALF = _CHUNK // 2
_NCH = _TOKW // _CHUNK
_NH = 2 * _NCH
_LANES = 16

_mesh = plsc.VectorSubcoreMesh(core_axis_name="c", subcore_axis_name="s")


@functools.partial(
    pl.kernel,
    out_type=jax.ShapeDtypeStruct((_N, _H), jnp.float32),
    mesh=_mesh,
    scratch_types=[
        pltpu.VMEM((_NCH, _CHUNK), jnp.int32),
        pltpu.VMEM((_NH, _HALF), jnp.int32),
        pltpu.VMEM((_CHUNK, _H), jnp.float32),
        pltpu.VMEM((_CHUNK, _H), jnp.float32),
        pltpu.VMEM((_HALF, _H), jnp.float32),
        pltpu.VMEM((_HALF, _H), jnp.float32),
        pltpu.SemaphoreType.DMA,
        pltpu.SemaphoreType.DMA,
        pltpu.SemaphoreType.DMA,
        pltpu.SemaphoreType.DMA,
        pltpu.SemaphoreType.DMA,
        pltpu.SemaphoreType.DMA,
    ],
)
def _embed(ids_hbm, pos_hbm, wt_hbm, pt_hbm, out_hbm,
           widx, pidx, bufw0, bufw1, bufp0, bufp1,
           semgw0, semgw1, semgp0, semgp1, semst0, semst1):
    wid = lax.axis_index("s") * _NC + lax.axis_index("c")
    pltpu.sync_copy(ids_hbm.at[wid], widx)
    pltpu.sync_copy(pos_hbm.at[wid], pidx)

    wslots = ((bufw0, semgw0, semst0), (bufw1, semgw1, semst1))
    pslots = ((bufp0, semgp0), (bufp1, semgp1))

    def segment(c, k, mode):
        bufw, semgw, semst = wslots[k]
        o_bufw, o_semgw, o_semst = wslots[1 - k]

        pltpu.make_async_copy(wt_hbm.at[pl.ds(0, _CHUNK)], bufw, semgw).wait()

        if mode != "first":
            pltpu.make_async_copy(
                o_bufw, out_hbm.at[pl.ds(0, _CHUNK)], o_semst).wait()

        def fire_words():
            pltpu.async_copy(wt_hbm.at[widx.at[c + 1]], o_bufw, o_semgw)

        if mode == "loop":
            pl.when(c + 1 < _NCH)(fire_words)
        elif c + 1 < _NCH:
            fire_words()

        row0 = wid * _TOKW + c * _CHUNK

        for q in range(2):
            bufp, semgp = pslots[q]
            pltpu.make_async_copy(
                wt_hbm.at[pl.ds(0, _HALF)], bufp, semgp).wait()

            @pl.loop(0, _HALF)
            def _rows(r):
                for j in range(_H // _LANES):
                    sl = pl.ds(j * _LANES, _LANES)
                    bufw[q * _HALF + r, sl] += bufp[r, sl]

            def fire_pos():
                pltpu.async_copy(
                    pt_hbm.at[pidx.at[2 * (c + 1) + q]], bufp, semgp)

            if mode == "loop":
                pl.when(c + 1 < _NCH)(fire_pos)
            elif c + 1 < _NCH:
                fire_pos()

            pltpu.async_copy(bufw.at[pl.ds(q * _HALF, _HALF)],
                             out_hbm.at[pl.ds(row0 + q * _HALF, _HALF)], semst)

    pltpu.async_copy(wt_hbm.at[widx.at[0]], bufw0, semgw0)
    pltpu.async_copy(pt_hbm.at[pidx.at[0]], bufp0, semgp0)
    pltpu.async_copy(pt_hbm.at[pidx.at[1]], bufp1, semgp1)

    segment(0, 0, mode="first")
    segment(1, 1, mode="second")

    @pl.loop(2, _NCH, step=2)
    def _ring(c0):
        for k in range(2):
            segment(c0 + k, k, mode="loop")

    pltpu.make_async_copy(bufw1, out_hbm.at[pl.ds(0, _CHUNK)], semst1).wait()


@jax.jit
def kernel(input_ids, position_ids, word_table, pos_table):
    ids = input_ids.astype(jnp.int32).reshape(_NW, _NCH, _CHUNK)
    pos = position_ids.astype(jnp.int32).reshape(_NW, _NH, _HALF)
    out = _embed(ids, pos, word_table, pos_table)
    return out.reshape(_B, _S, _H)


# speedup vs baseline: 1.6612x; 1.2878x over previous
import functools

import jax
import jax.numpy as jnp
from jax import lax
from jax.experimental import pallas as pl
from jax.experimental.pallas import tpu as pltpu
from jax.experimental.pallas import tpu_sc as plsc

_B, _S, _H = 4, 8192, 1024
_N = _B * _S
_NC, _NS = 2, 16
_NW = _NC * _NS
_TOKW = _N // _NW
_CHUNK = 16
_NCH = _TOKW // _CHUNK
_LANES = 16
_NPAIR = 3

_mesh = plsc.VectorSubcoreMesh(core_axis_name="c", subcore_axis_name="s")


@functools.partial(
    pl.kernel,
    out_type=jax.ShapeDtypeStruct((_N, _H), jnp.float32),
    mesh=_mesh,
    scratch_types=[
        pltpu.VMEM((_NCH, _CHUNK), jnp.int32),
        pltpu.VMEM((_NCH, _CHUNK), jnp.int32),
    ] + [pltpu.VMEM((_CHUNK, _H), jnp.float32)] * (2 * _NPAIR)
      + [pltpu.SemaphoreType.DMA] * (2 * _NPAIR),
)
def _embed(ids_hbm, pos_hbm, wt_hbm, pt_hbm, out_hbm,
           widx, pidx, bufw0, bufp0, bufw1, bufp1, bufw2, bufp2,
           semg0, semst0, semg1, semst1, semg2, semst2):
    wid = lax.axis_index("s") * _NC + lax.axis_index("c")
    pltpu.sync_copy(ids_hbm.at[wid], widx)
    pltpu.sync_copy(pos_hbm.at[wid], pidx)

    pairs = ((bufw0, bufp0, semg0, semst0),
             (bufw1, bufp1, semg1, semst1),
             (bufw2, bufp2, semg2, semst2))

    def segment(c, k, first):
        bufw, bufp, semg, semst = pairs[k]
        nbufw, nbufp, nsemg, nsemst = pairs[(k + 2) % _NPAIR]

        pltpu.make_async_copy(wt_hbm.at[pl.ds(0, _CHUNK)], bufw, semg).wait()
        pltpu.make_async_copy(wt_hbm.at[pl.ds(0, _CHUNK)], bufp, semg).wait()

        if first:
            pltpu.async_copy(wt_hbm.at[widx.at[2]], nbufw, nsemg)
            pltpu.async_copy(pt_hbm.at[pidx.at[2]], nbufp, nsemg)
        else:
            pltpu.make_async_copy(
                nbufw, out_hbm.at[pl.ds(0, _CHUNK)], nsemst).wait()

            @pl.when(c < _NCH - 2)
            def _fire_next():
                pltpu.async_copy(wt_hbm.at[widx.at[c + 2]], nbufw, nsemg)
                pltpu.async_copy(pt_hbm.at[pidx.at[c + 2]], nbufp, nsemg)

        row0 = wid * _TOKW + c * _CHUNK
        half = _CHUNK // 2
        for q in range(2):
            @pl.loop(q * half, (q + 1) * half)
            def _rows(r):
                for j in range(_H // _LANES):
                    sl = pl.ds(j * _LANES, _LANES)
                    bufw[r, sl] += bufp[r, sl]

            pltpu.async_copy(bufw.at[pl.ds(q * half, half)],
                             out_hbm.at[pl.ds(row0 + q * half, half)], semst)

    pltpu.async_copy(wt_hbm.at[widx.at[0]], bufw0, semg0)
    pltpu.async_copy(pt_hbm.at[pidx.at[0]], bufp0, semg0)
    pltpu.async_copy(wt_hbm.at[widx.at[1]], bufw1, semg1)
    pltpu.async_copy(pt_hbm.at[pidx.at[1]], bufp1, semg1)

    segment(0, 0, first=True)

    @pl.loop(1, _NCH, step=_NPAIR)
    def _ring(c0):
        for k in range(_NPAIR):
            segment(c0 + k, (1 + k) % _NPAIR, first=False)

    pltpu.make_async_copy(bufw0, out_hbm.at[pl.ds(0, _CHUNK)], semst0).wait()


@jax.jit
def kernel(input_ids, position_ids, word_table, pos_table):
    ids = input_ids.astype(jnp.int32).reshape(_NW, _NCH, _CHUNK)
    pos = position_ids.astype(jnp.int32).reshape(_NW, _NCH, _CHUNK)
    out = _embed(ids, pos, word_table, pos_table)
    return out.reshape(_B, _S, _H)
